# baseline (speedup 1.0000x reference)
import functools

import jax
import jax.numpy as jnp
from jax import lax
from jax.experimental import pallas as pl
from jax.experimental.pallas import tpu as pltpu
from jax.experimental.pallas import tpu_sc as plsc

N_NODES = 10000
D = 128
N_EDGES = 320000

NC = 2
NS = 16
NW = NC * NS
EPW = N_EDGES // NW
CH = 400
NCHUNK = EPW // CH
NPS = N_NODES // NS
RB = 2000
G = N_NODES // RB

@functools.cache
def _make_degree_hist():
    mesh = plsc.VectorSubcoreMesh(core_axis_name="c", subcore_axis_name="s")
    return pl.kernel(
        _degree_hist_body,
        mesh=mesh,
        compiler_params=pltpu.CompilerParams(
            needs_layout_passes=False, use_tc_tiling_on_sc=False),
        out_type=jax.ShapeDtypeStruct((NW, N_NODES), jnp.float32),
        scratch_types=[
            pltpu.VMEM((EPW,), jnp.int32),
            pltpu.VMEM((N_NODES,), jnp.float32),
        ],
    )


def _degree_hist_body(ei_hbm, out_hbm, dst_v, hist_v):
    cid = lax.axis_index("c")
    sid = lax.axis_index("s")
    wid = sid * NC + cid
    pltpu.sync_copy(ei_hbm.at[1, pl.ds(wid * EPW, EPW)], dst_v)

    zeros = jnp.zeros((16,), jnp.float32)

    def zero_body(i, _):
        hist_v[pl.ds(i * 16, 16)] = zeros
        return ()

    lax.fori_loop(0, N_NODES // 16, zero_body, ())

    ones = jnp.ones((16,), jnp.float32)

    def body(i, _):
        idx = dst_v[pl.ds(i * 16, 16)]
        plsc.addupdate_scatter(hist_v, [idx], ones)
        return ()

    lax.fori_loop(0, EPW // 16, body, ())
    pltpu.sync_copy(hist_v, out_hbm.at[wid])


@functools.cache
def _make_edge_aggregate():
    mesh = plsc.VectorSubcoreMesh(core_axis_name="c", subcore_axis_name="s")
    return pl.kernel(
        _edge_aggregate_body,
        mesh=mesh,
        compiler_params=pltpu.CompilerParams(
            needs_layout_passes=False, use_tc_tiling_on_sc=False),
        out_type=jax.ShapeDtypeStruct((NC, N_NODES, D), jnp.bfloat16),
        scratch_types=[
            pltpu.VMEM((EPW,), jnp.int32),
            pltpu.VMEM((EPW,), jnp.int32),
            pltpu.VMEM((CH, D), jnp.bfloat16),
            pltpu.VMEM((CH, D), jnp.bfloat16),
            pltpu.VMEM_SHARED((N_NODES, D), jnp.bfloat16),
            pltpu.SemaphoreType.DMA,
            pltpu.SemaphoreType.DMA,
        ],
    )


def _edge_aggregate_body(h_hbm, ei_hbm, out_hbm,
                         src_v, dst_v, rows_a, rows_b, acc_sh,
                         sem_ga, sem_gb):
    cid = lax.axis_index("c")
    sid = lax.axis_index("s")
    wid = sid * NC + cid

    pltpu.sync_copy(ei_hbm.at[0, pl.ds(wid * EPW, EPW)], src_v)
    pltpu.sync_copy(ei_hbm.at[1, pl.ds(wid * EPW, EPW)], dst_v)

    zeros = jnp.zeros((32,), jnp.bfloat16)

    def zero_body(i, _):
        rows_a[i // 4, pl.ds((i % 4) * 32, 32)] = zeros
        return ()

    lax.fori_loop(0, CH * (D // 32), zero_body, ())
    for k in range(NPS // CH):
        pltpu.sync_copy(rows_a, acc_sh.at[pl.ds(sid * NPS + k * CH, CH)])
    tail = NPS - (NPS // CH) * CH
    if tail:
        pltpu.sync_copy(rows_a.at[pl.ds(0, tail)],
                        acc_sh.at[pl.ds(sid * NPS + (NPS // CH) * CH, tail)])
    plsc.subcore_barrier()

    def sidx(j):
        return src_v.at[pl.ds(j * CH, CH)]

    def didx(j):
        return dst_v.at[pl.ds(j * CH, CH)]

    def g_start(j, buf, sem):
        pltpu.async_copy(h_hbm.at[sidx(j)], buf, sem)

    def g_wait(buf, sem):
        pltpu.make_async_copy(h_hbm.at[sidx(0)], buf, sem).wait()

    def s_sync(j, buf):
        pltpu.sync_copy(buf, acc_sh.at[didx(j)], add=True)

    g_start(0, rows_a, sem_ga)

    def chunk_pair(i, _):
        j0 = i * 2
        g_start(j0 + 1, rows_b, sem_gb)
        g_wait(rows_a, sem_ga)
        s_sync(j0, rows_a)
        g_start(j0 + 2, rows_a, sem_ga)
        g_wait(rows_b, sem_gb)
        s_sync(j0 + 1, rows_b)
        return ()

    lax.fori_loop(0, (NCHUNK - 1) // 2, chunk_pair, ())
    g_wait(rows_a, sem_ga)
    s_sync(NCHUNK - 1, rows_a)
    plsc.subcore_barrier()
    pltpu.sync_copy(acc_sh.at[pl.ds(sid * NPS, NPS)],
                    out_hbm.at[cid, pl.ds(sid * NPS, NPS)])


def _tc1_body(hist_ref, x_ref, w_ref, dinv_ref, hs_ref):
    deg = jnp.sum(hist_ref[...], axis=1, keepdims=True) + 1.0
    dinv = lax.rsqrt(deg)
    h = jnp.dot(x_ref[...], w_ref[...], preferred_element_type=jnp.float32)
    dinv_ref[...] = dinv
    hs_ref[...] = (h * dinv).astype(jnp.bfloat16)


_tc1 = pl.pallas_call(
    _tc1_body,
    grid=(G,),
    in_specs=[
        pl.BlockSpec((RB, NW), lambda i: (i, 0)),
        pl.BlockSpec((RB, D), lambda i: (i, 0)),
        pl.BlockSpec((D, D), lambda i: (0, 0)),
    ],
    out_specs=[
        pl.BlockSpec((RB, 1), lambda i: (i, 0)),
        pl.BlockSpec((RB, D), lambda i: (i, 0)),
    ],
    out_shape=[
        jax.ShapeDtypeStruct((N_NODES, 1), jnp.float32),
        jax.ShapeDtypeStruct((N_NODES, D), jnp.bfloat16),
    ],
)


def _tc2_body(agg_ref, hs_ref, dinv_ref, b_ref, w_ref, out_ref):
    dinv = dinv_ref[...]
    s = (agg_ref[0].astype(jnp.float32) + agg_ref[1].astype(jnp.float32)
         + hs_ref[...].astype(jnp.float32))
    z = jnp.maximum(dinv * s + b_ref[...], 0.0)
    h2 = jnp.dot(z, w_ref[...], preferred_element_type=jnp.float32)
    out_ref[...] = (h2 * dinv).astype(jnp.bfloat16)


_tc2 = pl.pallas_call(
    _tc2_body,
    grid=(G,),
    in_specs=[
        pl.BlockSpec((NC, RB, D), lambda i: (0, i, 0)),
        pl.BlockSpec((RB, D), lambda i: (i, 0)),
        pl.BlockSpec((RB, 1), lambda i: (i, 0)),
        pl.BlockSpec((1, D), lambda i: (0, 0)),
        pl.BlockSpec((D, D), lambda i: (0, 0)),
    ],
    out_specs=pl.BlockSpec((RB, D), lambda i: (i, 0)),
    out_shape=jax.ShapeDtypeStruct((N_NODES, D), jnp.bfloat16),
)


def _tc3_body(agg_ref, hs_ref, dinv_ref, b_ref, out_ref):
    dinv = dinv_ref[...]
    s = (agg_ref[0].astype(jnp.float32) + agg_ref[1].astype(jnp.float32)
         + hs_ref[...].astype(jnp.float32))
    out_ref[...] = dinv * s + b_ref[...]


_tc3 = pl.pallas_call(
    _tc3_body,
    grid=(G,),
    in_specs=[
        pl.BlockSpec((NC, RB, D), lambda i: (0, i, 0)),
        pl.BlockSpec((RB, D), lambda i: (i, 0)),
        pl.BlockSpec((RB, 1), lambda i: (i, 0)),
        pl.BlockSpec((1, D), lambda i: (0, 0)),
    ],
    out_specs=pl.BlockSpec((RB, D), lambda i: (i, 0)),
    out_shape=jax.ShapeDtypeStruct((N_NODES, D), jnp.float32),
)


def kernel(x, edge_index, W1, b1, W2, b2):
    ei = edge_index.astype(jnp.int32)

    degree_hist = _make_degree_hist()
    edge_aggregate = _make_edge_aggregate()

    hist = degree_hist(ei)
    dinv, hs1 = _tc1(hist.T, x, W1)
    agg1 = edge_aggregate(hs1, ei)
    hs2 = _tc2(agg1, hs1, dinv, b1.reshape(1, D), W2)
    agg2 = edge_aggregate(hs2, ei)
    out = _tc3(agg2, hs2, dinv, b2.reshape(1, D))
    return out


# speedup vs baseline: 1.0154x; 1.0154x over previous
import functools

import jax
import jax.numpy as jnp
from jax import lax
from jax.experimental import pallas as pl
from jax.experimental.pallas import tpu as pltpu
from jax.experimental.pallas import tpu_sc as plsc

N_NODES = 10000
D = 128
N_EDGES = 320000

NC = 2
NS = 16
NW = NC * NS
EPW = N_EDGES // NW
CH = 400
NCHUNK = EPW // CH
NPS = N_NODES // NS
RB = 2000
G = N_NODES // RB

@functools.cache
def _make_degree_hist():
    mesh = plsc.VectorSubcoreMesh(core_axis_name="c", subcore_axis_name="s")
    return pl.kernel(
        _degree_hist_body,
        mesh=mesh,
        compiler_params=pltpu.CompilerParams(
            needs_layout_passes=False, use_tc_tiling_on_sc=False),
        out_type=jax.ShapeDtypeStruct((NW, N_NODES), jnp.float32),
        scratch_types=[
            pltpu.VMEM((EPW,), jnp.int32),
            pltpu.VMEM((N_NODES,), jnp.float32),
        ],
    )


def _degree_hist_body(ei_hbm, out_hbm, dst_v, hist_v):
    cid = lax.axis_index("c")
    sid = lax.axis_index("s")
    wid = sid * NC + cid
    pltpu.sync_copy(ei_hbm.at[1, pl.ds(wid * EPW, EPW)], dst_v)

    zeros = jnp.zeros((16,), jnp.float32)

    def zero_body(i, _):
        hist_v[pl.ds(i * 16, 16)] = zeros
        return ()

    lax.fori_loop(0, N_NODES // 16, zero_body, ())

    ones = jnp.ones((16,), jnp.float32)

    def body(i, _):
        idx = dst_v[pl.ds(i * 16, 16)]
        plsc.addupdate_scatter(hist_v, [idx], ones)
        return ()

    lax.fori_loop(0, EPW // 16, body, ())
    pltpu.sync_copy(hist_v, out_hbm.at[wid])


@functools.cache
def _make_edge_aggregate():
    mesh = plsc.VectorSubcoreMesh(core_axis_name="c", subcore_axis_name="s")
    return pl.kernel(
        _edge_aggregate_body,
        mesh=mesh,
        compiler_params=pltpu.CompilerParams(
            needs_layout_passes=False, use_tc_tiling_on_sc=False),
        out_type=jax.ShapeDtypeStruct((NC, N_NODES, D), jnp.bfloat16),
        scratch_types=[
            pltpu.VMEM((EPW,), jnp.int32),
            pltpu.VMEM((EPW,), jnp.int32),
            pltpu.VMEM((CH, D), jnp.bfloat16),
            pltpu.VMEM((CH, D), jnp.bfloat16),
            pltpu.VMEM_SHARED((N_NODES, D), jnp.bfloat16),
            pltpu.SemaphoreType.DMA,
            pltpu.SemaphoreType.DMA,
        ],
    )


def _edge_aggregate_body(h_hbm, ei_hbm, out_hbm,
                         src_v, dst_v, rows_a, rows_b, acc_sh,
                         sem_ga, sem_gb):
    cid = lax.axis_index("c")
    sid = lax.axis_index("s")
    wid = sid * NC + cid

    def sidx(j):
        return src_v.at[pl.ds(j * CH, CH)]

    def didx(j):
        return dst_v.at[pl.ds(j * CH, CH)]

    def g_start(j, buf, sem):
        pltpu.async_copy(h_hbm.at[sidx(j)], buf, sem)

    def g_wait(buf, sem):
        pltpu.make_async_copy(h_hbm.at[sidx(0)], buf, sem).wait()

    def s_sync(j, buf):
        pltpu.sync_copy(buf, acc_sh.at[didx(j)], add=True)

    pltpu.sync_copy(ei_hbm.at[0, pl.ds(wid * EPW, EPW)], src_v)
    g_start(0, rows_a, sem_ga)
    pltpu.sync_copy(ei_hbm.at[1, pl.ds(wid * EPW, EPW)], dst_v)

    zeros = jnp.zeros((32,), jnp.bfloat16)

    def zero_body(i, _):
        rows_b[i // 4, pl.ds((i % 4) * 32, 32)] = zeros
        return ()

    lax.fori_loop(0, CH * (D // 32), zero_body, ())
    for k in range(NPS // CH):
        pltpu.sync_copy(rows_b, acc_sh.at[pl.ds(sid * NPS + k * CH, CH)])
    tail = NPS - (NPS // CH) * CH
    if tail:
        pltpu.sync_copy(rows_b.at[pl.ds(0, tail)],
                        acc_sh.at[pl.ds(sid * NPS + (NPS // CH) * CH, tail)])
    plsc.subcore_barrier()


    def chunk_pair(i, _):
        j0 = i * 2
        g_start(j0 + 1, rows_b, sem_gb)
        g_wait(rows_a, sem_ga)
        s_sync(j0, rows_a)
        g_start(j0 + 2, rows_a, sem_ga)
        g_wait(rows_b, sem_gb)
        s_sync(j0 + 1, rows_b)
        return ()

    lax.fori_loop(0, (NCHUNK - 1) // 2, chunk_pair, ())
    g_wait(rows_a, sem_ga)
    s_sync(NCHUNK - 1, rows_a)
    plsc.subcore_barrier()
    pltpu.sync_copy(acc_sh.at[pl.ds(sid * NPS, NPS)],
                    out_hbm.at[cid, pl.ds(sid * NPS, NPS)])


def _tc1_body(hist_ref, x_ref, w_ref, dinv_ref, hs_ref):
    deg = jnp.sum(hist_ref[...], axis=1, keepdims=True) + 1.0
    dinv = lax.rsqrt(deg)
    h = jnp.dot(x_ref[...], w_ref[...], preferred_element_type=jnp.float32)
    dinv_ref[...] = dinv
    hs_ref[...] = (h * dinv).astype(jnp.bfloat16)


_tc1 = pl.pallas_call(
    _tc1_body,
    grid=(G,),
    in_specs=[
        pl.BlockSpec((RB, NW), lambda i: (i, 0)),
        pl.BlockSpec((RB, D), lambda i: (i, 0)),
        pl.BlockSpec((D, D), lambda i: (0, 0)),
    ],
    out_specs=[
        pl.BlockSpec((RB, 1), lambda i: (i, 0)),
        pl.BlockSpec((RB, D), lambda i: (i, 0)),
    ],
    out_shape=[
        jax.ShapeDtypeStruct((N_NODES, 1), jnp.float32),
        jax.ShapeDtypeStruct((N_NODES, D), jnp.bfloat16),
    ],
)


def _tc2_body(agg_ref, hs_ref, dinv_ref, b_ref, w_ref, out_ref):
    dinv = dinv_ref[...]
    s = (agg_ref[0].astype(jnp.float32) + agg_ref[1].astype(jnp.float32)
         + hs_ref[...].astype(jnp.float32))
    z = jnp.maximum(dinv * s + b_ref[...], 0.0)
    h2 = jnp.dot(z, w_ref[...], preferred_element_type=jnp.float32)
    out_ref[...] = (h2 * dinv).astype(jnp.bfloat16)


_tc2 = pl.pallas_call(
    _tc2_body,
    grid=(G,),
    in_specs=[
        pl.BlockSpec((NC, RB, D), lambda i: (0, i, 0)),
        pl.BlockSpec((RB, D), lambda i: (i, 0)),
        pl.BlockSpec((RB, 1), lambda i: (i, 0)),
        pl.BlockSpec((1, D), lambda i: (0, 0)),
        pl.BlockSpec((D, D), lambda i: (0, 0)),
    ],
    out_specs=pl.BlockSpec((RB, D), lambda i: (i, 0)),
    out_shape=jax.ShapeDtypeStruct((N_NODES, D), jnp.bfloat16),
)


def _tc3_body(agg_ref, hs_ref, dinv_ref, b_ref, out_ref):
    dinv = dinv_ref[...]
    s = (agg_ref[0].astype(jnp.float32) + agg_ref[1].astype(jnp.float32)
         + hs_ref[...].astype(jnp.float32))
    out_ref[...] = dinv * s + b_ref[...]


_tc3 = pl.pallas_call(
    _tc3_body,
    grid=(G,),
    in_specs=[
        pl.BlockSpec((NC, RB, D), lambda i: (0, i, 0)),
        pl.BlockSpec((RB, D), lambda i: (i, 0)),
        pl.BlockSpec((RB, 1), lambda i: (i, 0)),
        pl.BlockSpec((1, D), lambda i: (0, 0)),
    ],
    out_specs=pl.BlockSpec((RB, D), lambda i: (i, 0)),
    out_shape=jax.ShapeDtypeStruct((N_NODES, D), jnp.float32),
)


def kernel(x, edge_index, W1, b1, W2, b2):
    ei = edge_index.astype(jnp.int32)

    degree_hist = _make_degree_hist()
    edge_aggregate = _make_edge_aggregate()

    hist = degree_hist(ei)
    dinv, hs1 = _tc1(hist.T, x, W1)
    agg1 = edge_aggregate(hs1, ei)
    hs2 = _tc2(agg1, hs1, dinv, b1.reshape(1, D), W2)
    agg2 = edge_aggregate(hs2, ei)
    out = _tc3(agg2, hs2, dinv, b2.reshape(1, D))
    return out
